# Initial kernel scaffold; baseline (speedup 1.0000x reference)
import jax
import jax.numpy as jnp
from jax.experimental import pallas as pl


def kernel(x, pieces, ranks, files, tiles, zeros):
    raise NotImplementedError("write your pallas kernel here")


# speedup vs baseline: 5.2280x; 5.2280x over previous
import functools

import jax
import jax.numpy as jnp
from jax import lax
from jax.experimental import pallas as pl
from jax.experimental.pallas import tpu as pltpu
from jax.experimental.pallas import tpu_sc as plsc

DOUT = 128
BATCH = 16384
K = 32
NROWS = 768
CHUNK = 64


def _weight_body(t_ref, p_ref, r_ref, f_ref, o_ref):
    o_ref[...] = t_ref[...] + p_ref[...] + r_ref[...] + f_ref[...]


def _build_weight(pieces, ranks, files, tiles):
    shape4 = (12, 8, 8, DOUT)
    t2 = tiles.reshape(NROWS, DOUT)
    p2 = jnp.broadcast_to(pieces, shape4).reshape(NROWS, DOUT)
    r2 = jnp.broadcast_to(ranks, shape4).reshape(NROWS, DOUT)
    f2 = jnp.broadcast_to(files, shape4).reshape(NROWS, DOUT)
    return pl.pallas_call(
        _weight_body,
        out_shape=jax.ShapeDtypeStruct((NROWS, DOUT), jnp.float32),
    )(t2, p2, r2, f2)


@functools.cache
def _make_sc_kernel():
    nc, ns = 2, 16
    nw = nc * ns
    b_per_w = BATCH // nw
    n_chunks = b_per_w // CHUNK
    mesh = plsc.VectorSubcoreMesh(core_axis_name="c", subcore_axis_name="s")

    @functools.partial(
        pl.kernel,
        out_type=jax.ShapeDtypeStruct((BATCH * DOUT,), jnp.float32),
        mesh=mesh,
        scratch_types=[
            pltpu.VMEM(((NROWS + 1) * DOUT,), jnp.float32),
            pltpu.VMEM((CHUNK * K,), jnp.int32),
            pltpu.VMEM((CHUNK * DOUT,), jnp.float32),
        ],
        compiler_params=pltpu.CompilerParams(needs_layout_passes=False),
    )
    def sc_emb(w_hbm, z_hbm, x_hbm, out_hbm, table_v, x_v, o_v):
        wid = lax.axis_index("s") * nc + lax.axis_index("c")
        pltpu.sync_copy(w_hbm, table_v.at[pl.ds(0, NROWS * DOUT)])
        pltpu.sync_copy(z_hbm, table_v.at[pl.ds(NROWS * DOUT, DOUT)])
        base = wid * b_per_w
        cols = [lax.iota(jnp.int32, 16) + (16 * c) for c in range(8)]

        def chunk_body(ck, carry):
            row0 = base + ck * CHUNK
            pltpu.sync_copy(x_hbm.at[pl.ds(row0 * K, CHUNK * K)], x_v)

            def board_body(b, carry2):
                accs = [None] * 8
                for j in range(K):
                    idx_vec = jnp.full((16,), b * K + j, jnp.int32)
                    row = plsc.load_gather(x_v, [idx_vec])
                    addr = row * DOUT
                    for c in range(8):
                        val = plsc.load_gather(table_v, [addr + cols[c]])
                        accs[c] = val if j == 0 else accs[c] + val
                for c in range(8):
                    o_v[pl.ds(b * DOUT + 16 * c, 16)] = accs[c]
                return carry2

            lax.fori_loop(0, CHUNK, board_body, 0)
            pltpu.sync_copy(o_v, out_hbm.at[pl.ds(row0 * DOUT, CHUNK * DOUT)])
            return carry

        lax.fori_loop(0, n_chunks, chunk_body, 0)

    return sc_emb


def kernel(x, pieces, ranks, files, tiles, zeros):
    weight = _build_weight(pieces, ranks, files, tiles)
    out = _make_sc_kernel()(
        weight.reshape(-1), zeros.reshape(-1),
        x.astype(jnp.int32).reshape(-1))
    return out.reshape(BATCH, DOUT)
